# baseline (speedup 1.0000x reference)
import jax
import jax.numpy as jnp
from jax.experimental import pallas as pl
from jax.experimental.pallas import tpu as pltpu

_NQ = 4
_K = 1024
_D = 64


def _rvq_block_kernel(x_ref, cb_ref, cb3_ref, cn_ref, quant_ref, nn_ref,
                      counts_ref, loss_ref):
    j = pl.program_id(0)

    @pl.when(j == 0)
    def _init():
        counts_ref[...] = jnp.zeros_like(counts_ref)
        loss_ref[...] = jnp.zeros_like(loss_ref)

    x = x_ref[...]
    b = x.shape[0]
    r = x
    qsum = jnp.zeros_like(x)
    col_iota = jax.lax.broadcasted_iota(jnp.int32, (b, _K), 1)
    loss_sum = jnp.float32(0.0)
    nn_cols = []
    cnt_rows = []
    ones_row = jnp.ones((1, b), dtype=jnp.bfloat16)
    for i in range(_NQ):
        cb = cb_ref[i]
        cn = cn_ref[i:i + 1, :]
        s2 = jax.lax.dot_general(-2.0 * r, cb, (((1,), (1,)), ((), ())),
                                 preferred_element_type=jnp.float32)
        rn = jnp.sum(r * r, axis=1, keepdims=True)
        d2 = (rn + s2) + cn
        m = jnp.min(d2, axis=1, keepdims=True)
        idx = jnp.min(jnp.where(d2 == m, col_iota, _K), axis=1,
                      keepdims=True)
        oh = col_iota == idx
        onehot = oh.astype(jnp.bfloat16)
        q3 = jax.lax.dot_general(onehot, cb3_ref[i],
                                 (((1,), (0,)), ((), ())),
                                 preferred_element_type=jnp.float32)
        q = (q3[:, :_D] + q3[:, _D:2 * _D]) + q3[:, 2 * _D:]
        q_st = r + (q - r)
        qsum = qsum + q_st
        diff = r - q
        e = diff * diff
        loss_sum = loss_sum + jnp.sum(jnp.mean(e + 0.25 * e, axis=1))
        nn_cols.append(idx)
        cnt_rows.append(jax.lax.dot_general(
            ones_row, onehot, (((1,), (0,)), ((), ())),
            preferred_element_type=jnp.float32))
        r = r - q_st
    quant_ref[...] = qsum
    nn_ref[...] = jnp.concatenate(nn_cols, axis=1)
    counts_ref[...] += jnp.concatenate(cnt_rows, axis=0)
    loss_ref[...] += loss_sum.reshape(1, 1)


def kernel(inputs, codebooks):
    shape = inputs.shape
    d = shape[-1]
    flat = inputs.reshape(-1, d)
    n = flat.shape[0]
    nq, k, _ = codebooks.shape
    cnorm = jnp.stack(
        [jnp.sum(codebooks[i] * codebooks[i], axis=1) for i in range(nq)],
        axis=0)
    mask = jnp.uint32(0xFFFF0000)
    u = codebooks
    hi = jax.lax.bitcast_convert_type(
        jax.lax.bitcast_convert_type(u, jnp.uint32) & mask, jnp.float32)
    r1 = u - hi
    mid = jax.lax.bitcast_convert_type(
        jax.lax.bitcast_convert_type(r1, jnp.uint32) & mask, jnp.float32)
    lo = r1 - mid
    cb3 = jnp.concatenate([hi, mid, lo], axis=-1).astype(jnp.bfloat16)
    blk = 1024
    grid = (n // blk,)
    quant, nn, counts, loss = pl.pallas_call(
        _rvq_block_kernel,
        grid=grid,
        in_specs=[
            pl.BlockSpec((blk, d), lambda j: (j, 0)),
            pl.BlockSpec((nq, k, d), lambda j: (0, 0, 0)),
            pl.BlockSpec((nq, k, 3 * d), lambda j: (0, 0, 0)),
            pl.BlockSpec((nq, k), lambda j: (0, 0)),
        ],
        out_specs=[
            pl.BlockSpec((blk, d), lambda j: (j, 0)),
            pl.BlockSpec((blk, nq), lambda j: (j, 0)),
            pl.BlockSpec((nq, k), lambda j: (0, 0)),
            pl.BlockSpec((1, 1), lambda j: (0, 0)),
        ],
        out_shape=[
            jax.ShapeDtypeStruct((n, d), jnp.float32),
            jax.ShapeDtypeStruct((n, nq), jnp.int32),
            jax.ShapeDtypeStruct((nq, k), jnp.float32),
            jax.ShapeDtypeStruct((1, 1), jnp.float32),
        ],
        compiler_params=pltpu.CompilerParams(
            dimension_semantics=("arbitrary",)),
    )(flat, codebooks, cb3, cnorm)
    quantized = quant.reshape(shape)
    qloss = loss[0, 0] / jnp.float32(n)
    qloss_out = jnp.full(shape[:-1] + (1,), qloss, dtype=jnp.float32)
    nn_idx = nn.T.reshape((nq,) + shape[:-1])
    codebooks_out = codebooks.reshape(-1, d)
    return quantized, qloss_out, nn_idx, codebooks_out, counts.astype(jnp.int32)


# speedup vs baseline: 1.0831x; 1.0831x over previous
import jax
import jax.numpy as jnp
from jax.experimental import pallas as pl
from jax.experimental.pallas import tpu as pltpu

_NQ = 4
_K = 1024
_D = 64


def _rvq_block_kernel(x_ref, cb_ref, cb3_ref, cn_ref, quant_ref, nn_ref,
                      counts_ref, loss_ref):
    j = pl.program_id(0)

    @pl.when(j == 0)
    def _init():
        counts_ref[...] = jnp.zeros_like(counts_ref)
        loss_ref[...] = jnp.zeros_like(loss_ref)

    x = x_ref[...]
    b = x.shape[0]
    r = x
    qsum = jnp.zeros_like(x)
    col_iota = jax.lax.broadcasted_iota(jnp.int32, (b, _K), 1)
    loss_sum = jnp.float32(0.0)
    nn_cols = []
    cnt_rows = []
    ones_row = jnp.ones((1, b), dtype=jnp.bfloat16)
    for i in range(_NQ):
        cb = cb_ref[i]
        cn = cn_ref[i:i + 1, :]
        s2 = jax.lax.dot_general(-2.0 * r, cb, (((1,), (1,)), ((), ())),
                                 preferred_element_type=jnp.float32)
        rn = jnp.sum(r * r, axis=1, keepdims=True)
        d2 = (rn + s2) + cn
        m = jnp.min(d2, axis=1, keepdims=True)
        oh0 = d2 == m
        ohb = oh0.astype(jnp.bfloat16)

        def _from_mask(maskb):
            p = jax.lax.dot_general(maskb, cb3_ref[i],
                                    (((1,), (0,)), ((), ())),
                                    preferred_element_type=jnp.float32)
            qv = (p[:, :_D] + p[:, _D:2 * _D]) + p[:, 2 * _D:3 * _D]
            idxv = (p[:, 3 * _D:3 * _D + 1]
                    + p[:, 3 * _D + 1:3 * _D + 2]).astype(jnp.int32)
            multv = p[:, 3 * _D + 2:3 * _D + 3]
            cntv = jax.lax.dot_general(ones_row, maskb,
                                       (((1,), (0,)), ((), ())),
                                       preferred_element_type=jnp.float32)
            return qv, idxv, multv, cntv

        q, idx, mult, cnt = _from_mask(ohb)

        def _tie_fix(_):
            idx1 = jnp.min(jnp.where(oh0, col_iota, _K), axis=1,
                           keepdims=True)
            oh1 = (col_iota == idx1).astype(jnp.bfloat16)
            q1, _, _, cnt1 = _from_mask(oh1)
            return q1, idx1, cnt1

        q, idx, cnt = jax.lax.cond(jnp.max(mult) > 1.5, _tie_fix,
                                   lambda _: (q, idx, cnt), None)
        q_st = r + (q - r)
        qsum = qsum + q_st
        diff = r - q
        e = diff * diff
        loss_sum = loss_sum + jnp.sum(jnp.mean(e + 0.25 * e, axis=1))
        nn_cols.append(idx)
        cnt_rows.append(cnt)
        r = r - q_st
    quant_ref[...] = qsum
    nn_ref[...] = jnp.concatenate(nn_cols, axis=1)
    counts_ref[...] += jnp.concatenate(cnt_rows, axis=0)
    loss_ref[...] += loss_sum.reshape(1, 1)


def kernel(inputs, codebooks):
    shape = inputs.shape
    d = shape[-1]
    flat = inputs.reshape(-1, d)
    n = flat.shape[0]
    nq, k, _ = codebooks.shape
    cnorm = jnp.stack(
        [jnp.sum(codebooks[i] * codebooks[i], axis=1) for i in range(nq)],
        axis=0)
    mask = jnp.uint32(0xFFFF0000)
    u = codebooks
    hi = jax.lax.bitcast_convert_type(
        jax.lax.bitcast_convert_type(u, jnp.uint32) & mask, jnp.float32)
    r1 = u - hi
    mid = jax.lax.bitcast_convert_type(
        jax.lax.bitcast_convert_type(r1, jnp.uint32) & mask, jnp.float32)
    lo = r1 - mid
    iota = jnp.arange(k, dtype=jnp.int32)
    extra = jnp.stack([(iota & ~3).astype(jnp.float32),
                       (iota & 3).astype(jnp.float32),
                       jnp.ones((k,), jnp.float32)], axis=1)
    cb3 = jnp.concatenate(
        [hi, mid, lo, jnp.broadcast_to(extra[None], (nq, k, 3))],
        axis=-1).astype(jnp.bfloat16)
    blk = 512
    grid = (n // blk,)
    quant, nn, counts, loss = pl.pallas_call(
        _rvq_block_kernel,
        grid=grid,
        in_specs=[
            pl.BlockSpec((blk, d), lambda j: (j, 0)),
            pl.BlockSpec((nq, k, d), lambda j: (0, 0, 0)),
            pl.BlockSpec((nq, k, 3 * d + 3), lambda j: (0, 0, 0)),
            pl.BlockSpec((nq, k), lambda j: (0, 0)),
        ],
        out_specs=[
            pl.BlockSpec((blk, d), lambda j: (j, 0)),
            pl.BlockSpec((blk, nq), lambda j: (j, 0)),
            pl.BlockSpec((nq, k), lambda j: (0, 0)),
            pl.BlockSpec((1, 1), lambda j: (0, 0)),
        ],
        out_shape=[
            jax.ShapeDtypeStruct((n, d), jnp.float32),
            jax.ShapeDtypeStruct((n, nq), jnp.int32),
            jax.ShapeDtypeStruct((nq, k), jnp.float32),
            jax.ShapeDtypeStruct((1, 1), jnp.float32),
        ],
        compiler_params=pltpu.CompilerParams(
            dimension_semantics=("arbitrary",)),
    )(flat, codebooks, cb3, cnorm)
    quantized = quant.reshape(shape)
    qloss = loss[0, 0] / jnp.float32(n)
    qloss_out = jnp.full(shape[:-1] + (1,), qloss, dtype=jnp.float32)
    nn_idx = nn.T.reshape((nq,) + shape[:-1])
    codebooks_out = codebooks.reshape(-1, d)
    return quantized, qloss_out, nn_idx, codebooks_out, counts.astype(jnp.int32)


# speedup vs baseline: 1.2278x; 1.1336x over previous
import jax
import jax.numpy as jnp
from jax.experimental import pallas as pl
from jax.experimental.pallas import tpu as pltpu

_NQ = 4
_K = 1024
_D = 64


def _rvq_block_kernel(x_ref, cb_ref, cb3_ref, cn_ref, quant_ref, nn_ref,
                      counts_ref, loss_ref):
    j = pl.program_id(0)

    @pl.when(j == 0)
    def _init():
        counts_ref[...] = jnp.zeros_like(counts_ref)
        loss_ref[...] = jnp.zeros_like(loss_ref)

    b2 = x_ref.shape[0]
    b = b2 // 2
    col_iota = jax.lax.broadcasted_iota(jnp.int32, (b, _K), 1)
    ones_row = jnp.ones((1, b), dtype=jnp.bfloat16)

    def _from_mask(maskb, i):
        p = jax.lax.dot_general(maskb, cb3_ref[i],
                                (((1,), (0,)), ((), ())),
                                preferred_element_type=jnp.float32)
        qv = (p[:, :_D] + p[:, _D:2 * _D]) + p[:, 2 * _D:3 * _D]
        idxv = (p[:, 3 * _D:3 * _D + 1]
                + p[:, 3 * _D + 1:3 * _D + 2]).astype(jnp.int32)
        multv = p[:, 3 * _D + 2:3 * _D + 3]
        cntv = jax.lax.dot_general(ones_row, maskb,
                                   (((1,), (0,)), ((), ())),
                                   preferred_element_type=jnp.float32)
        return qv, idxv, multv, cntv

    def _level(r, i):
        cb = cb_ref[i]
        cn = cn_ref[i:i + 1, :]
        s2 = jax.lax.dot_general(-2.0 * r, cb, (((1,), (1,)), ((), ())),
                                 preferred_element_type=jnp.float32)
        rn = jnp.sum(r * r, axis=1, keepdims=True)
        d2 = (rn + s2) + cn
        m = jnp.min(d2, axis=1, keepdims=True)
        oh0 = d2 == m
        q, idx, mult, cnt = _from_mask(oh0.astype(jnp.bfloat16), i)
        return q, idx, mult, cnt, oh0

    ra = x_ref[:b, :]
    rb = x_ref[b:, :]
    qsum_a = jnp.zeros_like(ra)
    qsum_b = jnp.zeros_like(rb)
    loss_sum = jnp.float32(0.0)
    nn_a, nn_b, cnt_rows = [], [], []
    for i in range(_NQ):
        q_a, idx_a, mult_a, cnt_a, oh_a = _level(ra, i)
        q_b, idx_b, mult_b, cnt_b, oh_b = _level(rb, i)

        def _tie_fix(_):
            ia = jnp.min(jnp.where(oh_a, col_iota, _K), axis=1, keepdims=True)
            ib = jnp.min(jnp.where(oh_b, col_iota, _K), axis=1, keepdims=True)
            qa, _, _, ca = _from_mask((col_iota == ia).astype(jnp.bfloat16), i)
            qb, _, _, cb_ = _from_mask((col_iota == ib).astype(jnp.bfloat16), i)
            return qa, ia, ca, qb, ib, cb_

        any_tie = jnp.maximum(jnp.max(mult_a), jnp.max(mult_b)) > 1.5
        q_a, idx_a, cnt_a, q_b, idx_b, cnt_b = jax.lax.cond(
            any_tie, _tie_fix,
            lambda _: (q_a, idx_a, cnt_a, q_b, idx_b, cnt_b), None)

        qst_a = ra + (q_a - ra)
        qst_b = rb + (q_b - rb)
        qsum_a = qsum_a + qst_a
        qsum_b = qsum_b + qst_b
        da = ra - q_a
        db = rb - q_b
        ea = da * da
        eb = db * db
        loss_sum = (loss_sum + jnp.sum(jnp.mean(ea + 0.25 * ea, axis=1))
                    + jnp.sum(jnp.mean(eb + 0.25 * eb, axis=1)))
        nn_a.append(idx_a)
        nn_b.append(idx_b)
        cnt_rows.append(cnt_a + cnt_b)
        ra = ra - qst_a
        rb = rb - qst_b
    quant_ref[:b, :] = qsum_a
    quant_ref[b:, :] = qsum_b
    nn_ref[:b, :] = jnp.concatenate(nn_a, axis=1)
    nn_ref[b:, :] = jnp.concatenate(nn_b, axis=1)
    counts_ref[...] += jnp.concatenate(cnt_rows, axis=0)
    loss_ref[...] += loss_sum.reshape(1, 1)


def kernel(inputs, codebooks):
    shape = inputs.shape
    d = shape[-1]
    flat = inputs.reshape(-1, d)
    n = flat.shape[0]
    nq, k, _ = codebooks.shape
    cnorm = jnp.stack(
        [jnp.sum(codebooks[i] * codebooks[i], axis=1) for i in range(nq)],
        axis=0)
    mask = jnp.uint32(0xFFFF0000)
    u = codebooks
    hi = jax.lax.bitcast_convert_type(
        jax.lax.bitcast_convert_type(u, jnp.uint32) & mask, jnp.float32)
    r1 = u - hi
    mid = jax.lax.bitcast_convert_type(
        jax.lax.bitcast_convert_type(r1, jnp.uint32) & mask, jnp.float32)
    lo = r1 - mid
    iota = jnp.arange(k, dtype=jnp.int32)
    extra = jnp.stack([(iota & ~3).astype(jnp.float32),
                       (iota & 3).astype(jnp.float32),
                       jnp.ones((k,), jnp.float32)], axis=1)
    cb3 = jnp.concatenate(
        [hi, mid, lo, jnp.broadcast_to(extra[None], (nq, k, 3))],
        axis=-1).astype(jnp.bfloat16)
    blk = 1024
    grid = (n // blk,)
    quant, nn, counts, loss = pl.pallas_call(
        _rvq_block_kernel,
        grid=grid,
        in_specs=[
            pl.BlockSpec((blk, d), lambda j: (j, 0)),
            pl.BlockSpec((nq, k, d), lambda j: (0, 0, 0)),
            pl.BlockSpec((nq, k, 3 * d + 3), lambda j: (0, 0, 0)),
            pl.BlockSpec((nq, k), lambda j: (0, 0)),
        ],
        out_specs=[
            pl.BlockSpec((blk, d), lambda j: (j, 0)),
            pl.BlockSpec((blk, nq), lambda j: (j, 0)),
            pl.BlockSpec((nq, k), lambda j: (0, 0)),
            pl.BlockSpec((1, 1), lambda j: (0, 0)),
        ],
        out_shape=[
            jax.ShapeDtypeStruct((n, d), jnp.float32),
            jax.ShapeDtypeStruct((n, nq), jnp.int32),
            jax.ShapeDtypeStruct((nq, k), jnp.float32),
            jax.ShapeDtypeStruct((1, 1), jnp.float32),
        ],
        compiler_params=pltpu.CompilerParams(
            dimension_semantics=("arbitrary",)),
    )(flat, codebooks, cb3, cnorm)
    quantized = quant.reshape(shape)
    qloss = loss[0, 0] / jnp.float32(n)
    qloss_out = jnp.full(shape[:-1] + (1,), qloss, dtype=jnp.float32)
    nn_idx = nn.T.reshape((nq,) + shape[:-1])
    codebooks_out = codebooks.reshape(-1, d)
    return quantized, qloss_out, nn_idx, codebooks_out, counts.astype(jnp.int32)
